# Initial kernel scaffold; baseline (speedup 1.0000x reference)
import jax
import jax.numpy as jnp
from jax.experimental import pallas as pl


def kernel(v, input_idx, input_sign):
    raise NotImplementedError("write your pallas kernel here")


# speedup vs baseline: 9.4840x; 9.4840x over previous
import functools

import jax
import jax.numpy as jnp
from jax import lax
from jax.experimental import pallas as pl
from jax.experimental.pallas import tpu as pltpu
from jax.experimental.pallas import tpu_sc as plsc

NC = 2
NS = 16
NW = NC * NS
LANES = 16
CHF = 1664


def _cdiv(a, b):
    return (a + b - 1) // b


def _make_table_builder(NP, RW):
    mesh = plsc.VectorSubcoreMesh(
        core_axis_name="c", subcore_axis_name="s", num_cores=NC,
        num_subcores=NS)

    @functools.partial(
        pl.kernel,
        out_type=jax.ShapeDtypeStruct((2 * NP, LANES), jnp.float32),
        mesh=mesh,
        scratch_types=[pltpu.VMEM((RW, LANES), jnp.float32)],
        compiler_params=pltpu.CompilerParams(use_tc_tiling_on_sc=False),
    )
    def build(vt_hbm, tbl_hbm, vbuf):
        wid = lax.axis_index("c") * NS + lax.axis_index("s")
        r0 = wid * RW
        pltpu.sync_copy(vt_hbm.at[pl.ds(r0, RW)], vbuf)
        pltpu.sync_copy(vbuf, tbl_hbm.at[pl.ds(r0, RW)])

        def nbody(r, carry):
            vbuf[r] = -vbuf[r]
            return carry

        lax.fori_loop(0, RW, nbody, 0)
        pltpu.sync_copy(vbuf, tbl_hbm.at[pl.ds(NP + r0, RW)])

    return build


def _make_main(NP, M_pad, per_worker):
    n_full = per_worker // CHF
    rem = per_worker - n_full * CHF

    mesh = plsc.VectorSubcoreMesh(
        core_axis_name="c", subcore_axis_name="s", num_cores=NC,
        num_subcores=NS)

    @functools.partial(
        pl.kernel,
        out_type=jax.ShapeDtypeStruct((M_pad, LANES), jnp.float32),
        mesh=mesh,
        scratch_types=[
            pltpu.VMEM((3, CHF), jnp.int32),
            pltpu.VMEM((3, CHF), jnp.float32),
            pltpu.VMEM((CHF, LANES), jnp.float32),
            pltpu.VMEM((CHF, LANES), jnp.float32),
            pltpu.VMEM((CHF, LANES), jnp.float32),
            pltpu.SemaphoreType.DMA,
        ],
        compiler_params=pltpu.CompilerParams(use_tc_tiling_on_sc=False),
    )
    def main(tbl, i0, i1, i2, s0, s1, s2, out, idxv, sgnv, b0, b1, b2, gsem):
        irefs = (i0, i1, i2)
        srefs = (s0, s1, s2)
        bufs = (b0, b1, b2)

        def process_chunk(base, ch):
            for k in range(3):
                pltpu.sync_copy(irefs[k].at[pl.ds(base, ch)],
                                idxv.at[k, pl.ds(0, ch)])
                pltpu.sync_copy(srefs[k].at[pl.ds(base, ch)],
                                sgnv.at[k, pl.ds(0, ch)])

            def ibody(g, carry):
                o = g * LANES
                for k in range(3):
                    ii = idxv[k, pl.ds(o, LANES)]
                    ss = sgnv[k, pl.ds(o, LANES)]
                    idxv[k, pl.ds(o, LANES)] = ii + jnp.where(
                        ss < 0.0, jnp.int32(NP), jnp.int32(0))
                return carry

            lax.fori_loop(0, ch // LANES, ibody, 0)

            descs = []
            for k in range(3):
                for j in range(ch // 128):
                    descs.append(pltpu.async_copy(
                        tbl.at[idxv.at[k, pl.ds(j * 128, 128)]],
                        bufs[k].at[pl.ds(j * 128, 128)],
                        gsem))
            for d in descs:
                d.wait()

            def cbody(i, carry):
                c = i * 4
                for u in range(4):
                    m = jnp.maximum(jnp.maximum(b0[c + u], b1[c + u]),
                                    b2[c + u])
                    b0[c + u] = 0.5 - 0.5 * m
                return carry

            lax.fori_loop(0, ch // 4, cbody, 0)
            pltpu.sync_copy(b0.at[pl.ds(0, ch)], out.at[pl.ds(base, ch)])

        wid = lax.axis_index("c") * NS + lax.axis_index("s")
        wbase = wid * per_worker

        def chunk_body(i, carry):
            process_chunk(wbase + i * CHF, CHF)
            return carry

        lax.fori_loop(0, n_full, chunk_body, 0)
        if rem:
            process_chunk(wbase + n_full * CHF, rem)

    return main


def kernel(v, input_idx, input_sign):
    B, N = v.shape
    M, K = input_idx.shape
    assert B == LANES and K == 3

    NP = _cdiv(N, NW * 8) * (NW * 8)
    RW = NP // NW
    per_worker = _cdiv(M, NW * 128) * 128
    M_pad = per_worker * NW

    vt = jnp.zeros((NP, LANES), jnp.float32).at[:N].set(v.T)
    idx_p = jnp.zeros((M_pad, 3), jnp.int32).at[:M].set(input_idx)
    sgn_p = jnp.ones((M_pad, 3), jnp.float32).at[:M].set(input_sign)

    tbl = _make_table_builder(NP, RW)(vt)
    outT = _make_main(NP, M_pad, per_worker)(
        tbl,
        idx_p[:, 0], idx_p[:, 1], idx_p[:, 2],
        sgn_p[:, 0], sgn_p[:, 1], sgn_p[:, 2])
    return outT[:M].T
